# Initial kernel scaffold; baseline (speedup 1.0000x reference)
import jax
import jax.numpy as jnp
from jax.experimental import pallas as pl


def kernel(features, memory_bank, min_val, max_val):
    raise NotImplementedError("write your pallas kernel here")


# speedup vs baseline: 8.5552x; 8.5552x over previous
import jax
import jax.numpy as jnp
from jax.experimental import pallas as pl
from jax.experimental.pallas import tpu as pltpu

_N = 1024
_D = 128
_K = 100000
_KB = 2048
_NSTEPS = (_K + _KB - 1) // _KB
_BIG = 3.0e38


def _knn_min_kernel(params_ref, f_ref, mb_ref, out_ref, acc_ref):
    k = pl.program_id(0)
    f = f_ref[...]
    fm2 = f * -2.0
    ones_row = jnp.ones((1, _D), jnp.float32)

    def partial_mins(masked):
        pm = None
        for j in range(_KB // 128):
            mbj = mb_ref[pl.ds(j * 128, 128), :]
            s = jax.lax.dot_general(
                fm2, mbj, (((1,), (1,)), ((), ())),
                preferred_element_type=jnp.float32)
            m2j = jax.lax.dot_general(
                ones_row, mbj * mbj, (((1,), (1,)), ((), ())),
                preferred_element_type=jnp.float32)
            part = s + m2j
            if masked:
                col = (k * _KB + j * 128
                       + jax.lax.broadcasted_iota(jnp.int32, (1, 128), 1))
                part = jnp.where(col < _K, part, _BIG)
            pm = part if pm is None else jnp.minimum(pm, part)
        return pm

    @pl.when(k == 0)
    def _():
        acc_ref[...] = jnp.full((_N, 128), _BIG, jnp.float32)

    @pl.when(k < _NSTEPS - 1)
    def _():
        acc_ref[...] = jnp.minimum(acc_ref[...], partial_mins(False))

    @pl.when(k == _NSTEPS - 1)
    def _():
        acc = jnp.minimum(acc_ref[...], partial_mins(True))
        f2 = jnp.sum(f * f, axis=1, keepdims=True)
        d2 = jnp.maximum(jnp.min(acc, axis=1, keepdims=True) + f2, 1e-12)
        d = jnp.sqrt(d2)
        mn = params_ref[0, 0]
        mx = params_ref[0, 1]
        out_ref[...] = (d - mn) / (mx - mn)


def kernel(features, memory_bank, min_val, max_val):
    params = jnp.stack([min_val, max_val]).reshape(1, 2)
    out = pl.pallas_call(
        _knn_min_kernel,
        grid=(_NSTEPS,),
        in_specs=[
            pl.BlockSpec(memory_space=pltpu.SMEM),
            pl.BlockSpec((_N, _D), lambda k: (0, 0)),
            pl.BlockSpec((_KB, _D), lambda k: (k, 0)),
        ],
        out_specs=pl.BlockSpec((_N, 1), lambda k: (0, 0)),
        out_shape=jax.ShapeDtypeStruct((_N, 1), jnp.float32),
        scratch_shapes=[pltpu.VMEM((_N, 128), jnp.float32)],
        compiler_params=pltpu.CompilerParams(
            dimension_semantics=("arbitrary",)),
    )(params, features, memory_bank)
    return out.reshape(_N)
